# Initial kernel scaffold; baseline (speedup 1.0000x reference)
import jax
import jax.numpy as jnp
from jax.experimental import pallas as pl


def kernel(input_node, input_edge, graph_index):
    raise NotImplementedError("write your pallas kernel here")


# speedup vs baseline: 17.9701x; 17.9701x over previous
import functools

import jax
import jax.numpy as jnp
from jax import lax
from jax.experimental import pallas as pl
from jax.experimental.pallas import tpu as pltpu
from jax.experimental.pallas import tpu_sc as plsc

_N_NODES = 10000
_N_EDGES = 320000
_NC = 2
_NS = 16
_L = 16
_NW = _NC * _NS
_EPW = _N_EDGES // _NW
_ITERS = _EPW // _L
_NPAD = 10240


def _sc_partials(nodes, edges, gi):
    mesh = plsc.VectorSubcoreMesh(core_axis_name="c", subcore_axis_name="s")

    @functools.partial(
        pl.kernel,
        out_type=jax.ShapeDtypeStruct((_NW, _NPAD), jnp.float32),
        mesh=mesh,
        compiler_params=pltpu.CompilerParams(needs_layout_passes=False),
        scratch_types=[
            pltpu.VMEM((_N_NODES,), jnp.float32),
            pltpu.VMEM((2 * _EPW,), jnp.int32),
            pltpu.VMEM((_EPW,), jnp.float32),
            pltpu.VMEM((_NPAD,), jnp.float32),
        ],
    )
    def k(nodes_hbm, edges_hbm, gi_hbm, out_hbm, nodes_v, gi_v, edge_v, acc_v):
        c = lax.axis_index("c")
        s = lax.axis_index("s")
        wid = s * _NC + c
        base = wid * _EPW

        pltpu.sync_copy(nodes_hbm, nodes_v)
        pltpu.sync_copy(gi_hbm.at[pl.ds(2 * base, 2 * _EPW)], gi_v)
        pltpu.sync_copy(edges_hbm.at[pl.ds(base, _EPW)], edge_v)

        zeros = jnp.zeros((_L,), jnp.float32)

        def zero_body(i, carry):
            acc_v[pl.ds(i * _L, _L)] = zeros
            return carry

        lax.fori_loop(0, _NPAD // _L, zero_body, 0)

        iota2 = lax.iota(jnp.int32, _L) * 2

        def body(j, carry):
            off = j * _L
            flat = 2 * off + iota2
            s_idx = plsc.load_gather(gi_v, [flat])
            r_idx = plsc.load_gather(gi_v, [flat + 1])
            e = edge_v[pl.ds(off, _L)]
            ns = plsc.load_gather(nodes_v, [s_idx])
            nr = plsc.load_gather(nodes_v, [r_idx])
            upd = (ns - nr) / e
            plsc.addupdate_scatter(acc_v, [r_idx], upd)
            return carry

        lax.fori_loop(0, _ITERS, body, 0)

        pltpu.sync_copy(acc_v, out_hbm.at[wid])

    return k(nodes, edges, gi)


def _tc_combine(partials):
    def body(p_ref, o_ref):
        o_ref[...] = jnp.sum(p_ref[...], axis=0, keepdims=True)

    return pl.pallas_call(
        body,
        out_shape=jax.ShapeDtypeStruct((1, _NPAD), jnp.float32),
    )(partials)


def kernel(input_node, input_edge, graph_index):
    nodes = input_node[:, 0]
    edges = input_edge[:, 0]
    partials = _sc_partials(nodes, edges, graph_index.reshape(-1))
    summed = _tc_combine(partials)
    return summed.reshape(-1)[:_N_NODES]
